# baseline (speedup 1.0000x reference)
import functools

import jax
import jax.numpy as jnp
from jax import lax
from jax.experimental import pallas as pl
from jax.experimental.pallas import tpu as pltpu
from jax.experimental.pallas import tpu_sc as plsc

_EPS = 1e-08
_NC, _NS, _L = 2, 16, 16
_NW = _NC * _NS


def _sc_householder_body(s, r, n_proc,
                         x_hbm, i0_hbm, i1_hbm, pn_hbm, out_hbm,
                         x_s, x_v, pn_s, pn_v, i0_v, i1_v, ns_v, sem):
    rp = r + 1
    shift = r.bit_length() - 1
    t_per_w = s // _NW
    n_groups = t_per_w // _L
    wid = lax.axis_index("c") * _NS + lax.axis_index("s")
    base = wid * t_per_w

    c_pn = pltpu.async_copy(pn_hbm, pn_s, sem)
    c_x = pltpu.async_copy(
        x_hbm.at[pl.ds(base * r, t_per_w * r)], x_s, sem)
    c_i0 = pltpu.async_copy(i0_hbm.at[pl.ds(base, t_per_w)], i0_v, sem)
    c_i1 = pltpu.async_copy(i1_hbm.at[pl.ds(base, t_per_w)], i1_v, sem)

    lanes = lax.iota(jnp.int32, _L)
    zero = jnp.zeros((_L,), jnp.float32)

    def repack_body(src, dst, i, carry):
        w = i * _L + lanes
        vals = plsc.load_gather(src, [w])
        plsc.store_scatter(dst, [w + (w >> shift)], vals)
        return carry

    c_pn.wait()
    lax.fori_loop(0, n_proc * r // _L,
                  functools.partial(repack_body, pn_s, pn_v), 0, unroll=8)
    for h in range(n_proc // _L):
        rowbase = (lanes + h * _L) * rp

        def norm_body(k, acc):
            vr = plsc.load_gather(pn_v, [rowbase + k])
            return acc + vr * vr

        ns_v[pl.ds(h * _L, _L)] = lax.fori_loop(
            0, r, norm_body, zero, unroll=8)

    c_x.wait()
    lax.fori_loop(0, t_per_w * r // _L,
                  functools.partial(repack_body, x_s, x_v), 0, unroll=8)
    c_i0.wait()
    c_i1.wait()

    for g in range(n_groups):
        b_x = (lanes + g * _L) * rp
        i0 = i0_v[pl.ds(g * _L, _L)]
        i1 = i1_v[pl.ds(g * _L, _L)]
        b_0 = i0 * rp
        b_1 = i1 * rp
        s0 = plsc.load_gather(ns_v, [i0]) + _EPS
        s1 = plsc.load_gather(ns_v, [i1]) + _EPS

        def dot_body(k, carry):
            a, b, d = carry
            v0 = plsc.load_gather(pn_v, [b_0 + k])
            v1 = plsc.load_gather(pn_v, [b_1 + k])
            xr = plsc.load_gather(x_v, [b_x + k])
            return (a + v0 * xr, b + v1 * xr, d + v0 * v1)

        a, b, d = lax.fori_loop(0, r, dot_body, (zero, zero, zero), unroll=8)
        c0 = (2.0 * a) / s0
        c1 = (2.0 * (b - c0 * d)) / s1

        def upd_body(k, carry):
            v0 = plsc.load_gather(pn_v, [b_0 + k])
            v1 = plsc.load_gather(pn_v, [b_1 + k])
            xr = plsc.load_gather(x_v, [b_x + k])
            plsc.store_scatter(x_v, [b_x + k], xr - c0 * v0 - c1 * v1)
            return carry

        lax.fori_loop(0, r, upd_body, 0, unroll=8)

    def unpack_body(i, carry):
        w = i * _L + lanes
        vals = plsc.load_gather(x_v, [w + (w >> shift)])
        plsc.store_scatter(x_s, [w], vals)
        return carry

    lax.fori_loop(0, t_per_w * r // _L, unpack_body, 0, unroll=8)
    pltpu.sync_copy(x_s, out_hbm.at[pl.ds(base * r, t_per_w * r)])


def _tc_proj_body(x2_ref, oi_ref, w_ref, out_ref):
    x2 = x2_ref[...]
    oi = oi_ref[...]
    s, r = x2.shape
    n_out = w_ref.shape[0] // r
    iota_e = lax.broadcasted_iota(jnp.int32, (s, n_out), 1)
    ohe = (oi == iota_e).astype(jnp.float32)
    xb = jnp.concatenate(
        [x2 * ohe[:, e:e + 1] for e in range(n_out)], axis=1)
    out_ref[...] = jnp.dot(xb, w_ref[...], preferred_element_type=jnp.float32)


def kernel(x, output_idx, process_indices, process_neurons, output_neurons):
    b, s, r = x.shape
    n_proc = process_neurons.shape[0]
    n_out, _, d_model = output_neurons.shape
    n_tok = b * s
    t_per_w = n_tok // _NW
    rp = r + 1

    xs = x.reshape(n_tok * r)
    pn = process_neurons.reshape(n_proc * r)
    oi = output_idx.reshape(n_tok, 1).astype(jnp.int32)
    pi0 = process_indices[..., 0].reshape(n_tok).astype(jnp.int32)
    pi1 = process_indices[..., 1].reshape(n_tok).astype(jnp.int32)
    w = output_neurons.reshape(n_out * r, d_model)

    sc_house = functools.partial(
        pl.kernel,
        out_type=jax.ShapeDtypeStruct((n_tok * r,), jnp.float32),
        mesh=plsc.VectorSubcoreMesh(core_axis_name="c", subcore_axis_name="s"),
        compiler_params=pltpu.CompilerParams(needs_layout_passes=False),
        scratch_types=[
            pltpu.VMEM((t_per_w * r,), jnp.float32),
            pltpu.VMEM((t_per_w * rp,), jnp.float32),
            pltpu.VMEM((n_proc * r,), jnp.float32),
            pltpu.VMEM((n_proc * rp,), jnp.float32),
            pltpu.VMEM((t_per_w,), jnp.int32),
            pltpu.VMEM((t_per_w,), jnp.int32),
            pltpu.VMEM((n_proc,), jnp.float32),
            pltpu.SemaphoreType.DMA,
        ],
    )(functools.partial(_sc_householder_body, n_tok, r, n_proc))
    x2 = sc_house(xs, pi0, pi1, pn).reshape(n_tok, r)

    out = pl.pallas_call(
        _tc_proj_body,
        out_shape=jax.ShapeDtypeStruct((n_tok, d_model), jnp.float32),
    )(x2, oi, w)
    return out.reshape(b, s, d_model)


# speedup vs baseline: 1.0243x; 1.0243x over previous
import functools

import jax
import jax.numpy as jnp
from jax import lax
from jax.experimental import pallas as pl
from jax.experimental.pallas import tpu as pltpu
from jax.experimental.pallas import tpu_sc as plsc

_EPS = 1e-08
_NC, _NS, _L = 2, 16, 16
_NW = _NC * _NS


def _sc_householder_body(s, r, n_proc,
                         x_hbm, i0_hbm, i1_hbm, pn_hbm, out_hbm,
                         x_v, pn_v, i0_v, i1_v, ns_v, sem):
    rp = r + 1
    t_per_w = s // _NW
    n_groups = t_per_w // _L
    wid = lax.axis_index("c") * _NS + lax.axis_index("s")
    base = wid * t_per_w

    c_pn = pltpu.async_copy(pn_hbm, pn_v, sem)
    c_x = pltpu.async_copy(
        x_hbm.at[pl.ds(base * rp, t_per_w * rp)], x_v, sem)
    c_i0 = pltpu.async_copy(i0_hbm.at[pl.ds(base, t_per_w)], i0_v, sem)
    c_i1 = pltpu.async_copy(i1_hbm.at[pl.ds(base, t_per_w)], i1_v, sem)

    lanes = lax.iota(jnp.int32, _L)
    zero = jnp.zeros((_L,), jnp.float32)

    c_pn.wait()
    for h in range(n_proc // _L):
        rowbase = (lanes + h * _L) * rp

        def norm_body(k, acc):
            vr = plsc.load_gather(pn_v, [rowbase + k])
            return acc + vr * vr

        ns_v[pl.ds(h * _L, _L)] = lax.fori_loop(
            0, r, norm_body, zero, unroll=8)

    c_x.wait()
    c_i0.wait()
    c_i1.wait()

    for g in range(n_groups):
        b_x = (lanes + g * _L) * rp
        i0 = i0_v[pl.ds(g * _L, _L)]
        i1 = i1_v[pl.ds(g * _L, _L)]
        b_0 = i0 * rp
        b_1 = i1 * rp
        s0 = plsc.load_gather(ns_v, [i0]) + _EPS
        s1 = plsc.load_gather(ns_v, [i1]) + _EPS

        def dot_body(k, carry):
            a, b, d = carry
            v0 = plsc.load_gather(pn_v, [b_0 + k])
            v1 = plsc.load_gather(pn_v, [b_1 + k])
            xr = plsc.load_gather(x_v, [b_x + k])
            return (a + v0 * xr, b + v1 * xr, d + v0 * v1)

        a, b, d = lax.fori_loop(0, r, dot_body, (zero, zero, zero), unroll=8)
        c0 = (2.0 * a) / s0
        c1 = (2.0 * (b - c0 * d)) / s1

        def upd_body(k, carry):
            v0 = plsc.load_gather(pn_v, [b_0 + k])
            v1 = plsc.load_gather(pn_v, [b_1 + k])
            xr = plsc.load_gather(x_v, [b_x + k])
            plsc.store_scatter(x_v, [b_x + k], xr - c0 * v0 - c1 * v1)
            return carry

        lax.fori_loop(0, r, upd_body, 0, unroll=8)

    pltpu.sync_copy(x_v, out_hbm.at[pl.ds(base * rp, t_per_w * rp)])


def _tc_proj_body(x2_ref, oi_ref, w_ref, out_ref):
    x2 = x2_ref[...][:, :-1]
    oi = oi_ref[...]
    s, r = x2.shape
    n_out = w_ref.shape[0] // r
    iota_e = lax.broadcasted_iota(jnp.int32, (s, n_out), 1)
    ohe = (oi == iota_e).astype(jnp.float32)
    xb = jnp.concatenate(
        [x2 * ohe[:, e:e + 1] for e in range(n_out)], axis=1)
    out_ref[...] = jnp.dot(xb, w_ref[...], preferred_element_type=jnp.float32)


def kernel(x, output_idx, process_indices, process_neurons, output_neurons):
    b, s, r = x.shape
    n_proc = process_neurons.shape[0]
    n_out, _, d_model = output_neurons.shape
    n_tok = b * s
    t_per_w = n_tok // _NW
    rp = r + 1

    xs = jnp.pad(x.reshape(n_tok, r), ((0, 0), (0, 1))).reshape(n_tok * rp)
    pn = jnp.pad(process_neurons, ((0, 0), (0, 1))).reshape(n_proc * rp)
    oi = output_idx.reshape(n_tok, 1).astype(jnp.int32)
    pi0 = process_indices[..., 0].reshape(n_tok).astype(jnp.int32)
    pi1 = process_indices[..., 1].reshape(n_tok).astype(jnp.int32)
    w = output_neurons.reshape(n_out * r, d_model)

    sc_house = functools.partial(
        pl.kernel,
        out_type=jax.ShapeDtypeStruct((n_tok * rp,), jnp.float32),
        mesh=plsc.VectorSubcoreMesh(core_axis_name="c", subcore_axis_name="s"),
        compiler_params=pltpu.CompilerParams(needs_layout_passes=False),
        scratch_types=[
            pltpu.VMEM((t_per_w * rp,), jnp.float32),
            pltpu.VMEM((n_proc * rp,), jnp.float32),
            pltpu.VMEM((t_per_w,), jnp.int32),
            pltpu.VMEM((t_per_w,), jnp.int32),
            pltpu.VMEM((n_proc,), jnp.float32),
            pltpu.SemaphoreType.DMA,
        ],
    )(functools.partial(_sc_householder_body, n_tok, r, n_proc))
    x2 = sc_house(xs, pi0, pi1, pn).reshape(n_tok, rp)

    out = pl.pallas_call(
        _tc_proj_body,
        out_shape=jax.ShapeDtypeStruct((n_tok, d_model), jnp.float32),
    )(x2, oi, w)
    return out.reshape(b, s, d_model)
